# Initial kernel scaffold; baseline (speedup 1.0000x reference)
import jax
import jax.numpy as jnp
from jax.experimental import pallas as pl


def kernel(x, adj_t, root_ptr, p, batch, group_ptr, Wl1, bl1, Wr1, Wl2, bl2, Wr2, Wlin, blin):
    raise NotImplementedError("write your pallas kernel here")


# speedup vs baseline: 2.5963x; 2.5963x over previous
import functools

import jax
import jax.numpy as jnp
from jax import lax
from jax.experimental import pallas as pl
from jax.experimental.pallas import tpu as pltpu
from jax.experimental.pallas import tpu_sc as plsc

N = 10000
E = 320000
D = 128
G = 256

NC = 2
NS = 16
LANES = 16
NW = NC * NS

NPAD = 10240
ROWS_PER_TILE = NPAD // NS
EPAD = 327680
EW = EPAD // NW
BATCH = 128
KB = 4
BLK_E = KB * BATCH
NBLK = EW // BLK_E

RB = 512
DW = 16


def _zero_vmem(ref, rows, cols):
  z = jnp.zeros((LANES,), jnp.float32)
  def body(r, carry):
    for c in range(cols // LANES):
      ref[r, pl.ds(c * LANES, LANES)] = z
    return carry
  lax.fori_loop(0, rows, body, 0)


def _fill_ones_vmem(ref, rows, cols):
  o = jnp.ones((LANES,), jnp.float32)
  def body(r, carry):
    for c in range(cols // LANES):
      ref[r, pl.ds(c * LANES, LANES)] = o
    return carry
  lax.fori_loop(0, rows, body, 0)


def _make_edge_kernel(with_deg):
  mesh = plsc.VectorSubcoreMesh(core_axis_name="c", subcore_axis_name="s")
  out_type = [jax.ShapeDtypeStruct((NC, NPAD, D), jnp.float32)]
  if with_deg:
    out_type.append(jax.ShapeDtypeStruct((NC, NPAD, D), jnp.float32))
  scratch = (
      [pltpu.VMEM((BATCH,), jnp.int32) for _ in range(KB)]
      + [pltpu.VMEM((BATCH,), jnp.int32) for _ in range(KB)]
      + [
          pltpu.VMEM((BATCH, D), jnp.float32),
          pltpu.VMEM_SHARED((NPAD, D), jnp.float32),
          pltpu.SemaphoreType.DMA,
      ]
  )

  def body(table, src2d, dst2d, agg_out, *rest):
    if with_deg:
      deg_out = rest[0]
      rest = rest[1:]
    sidx = rest[:KB]
    didx = rest[KB:2 * KB]
    (rows, agg_sh, sem) = rest[2 * KB:]
    cid = lax.axis_index("c")
    sid = lax.axis_index("s")
    wid = sid * NC + cid
    row0 = wid * (EW // BATCH)

    def zero_table():
      for k in range(ROWS_PER_TILE // BATCH):
        pltpu.sync_copy(rows, agg_sh.at[pl.ds(sid * ROWS_PER_TILE + k * BATCH, BATCH)])

    def dump_table(out):
      for k in range(ROWS_PER_TILE // BATCH):
        r = sid * ROWS_PER_TILE + k * BATCH
        pltpu.sync_copy(agg_sh.at[pl.ds(r, BATCH)], out.at[cid, pl.ds(r, BATCH)])

    if with_deg:
      _zero_vmem(rows, BATCH, D)
      zero_table()
      _fill_ones_vmem(rows, BATCH, D)
      plsc.subcore_barrier()

      def deg_block(b, carry):
        pltpu.sync_copy(dst2d.at[row0 + b], didx[0])
        pltpu.sync_copy(rows, agg_sh.at[didx[0]], add=True)
        return carry

      lax.fori_loop(0, EW // BATCH, deg_block, 0)
      plsc.subcore_barrier()
      dump_table(deg_out)
      plsc.subcore_barrier()

    _zero_vmem(rows, BATCH, D)
    zero_table()
    plsc.subcore_barrier()

    def block(b, carry):
      r = row0 + b * KB
      for j in range(KB):
        pltpu.sync_copy(src2d.at[r + j], sidx[j])
        pltpu.sync_copy(dst2d.at[r + j], didx[j])
      for j in range(KB):
        pltpu.async_copy(table.at[sidx[j]], rows, sem).wait()
        pltpu.sync_copy(rows, agg_sh.at[didx[j]], add=True)
      return carry

    lax.fori_loop(0, NBLK, block, 0)
    plsc.subcore_barrier()
    dump_table(agg_out)

  return pl.kernel(body, out_type=out_type, mesh=mesh, scratch_types=scratch)


_edge_deg = _make_edge_kernel(True)
_edge = _make_edge_kernel(False)


def _tc1_body(x_ref, wl_ref, wr_ref, bl_ref, u_ref, v_ref):
  xb = x_ref[...]
  u_ref[...] = jnp.dot(xb, wl_ref[...], preferred_element_type=jnp.float32)
  v_ref[...] = jnp.dot(xb, wr_ref[...], preferred_element_type=jnp.float32) + bl_ref[...]


def _tc2_body(s0, s1, d0, d1, v1, pcol, wl, wr, bl, u2, v2):
  deg = d0[...][:, :1] + d1[...][:, :1]
  rdeg = 1.0 / jnp.maximum(deg, 1.0)
  h1 = jnp.maximum((s0[...] + s1[...]) * rdeg + v1[...], 0.0)
  xs = h1 * pcol[...]
  u2[...] = jnp.dot(xs, wl[...], preferred_element_type=jnp.float32)
  v2[...] = jnp.dot(xs, wr[...], preferred_element_type=jnp.float32) + bl[...]


def _tc3_body(s0, s1, d0, d1, v2, pcol, batch_ref, root_ref, wla, wlb, blin_ref,
              out_ref, acc_r, acc_p, cnt):
  i = pl.program_id(0)

  @pl.when(i == 0)
  def _():
    acc_r[...] = jnp.zeros_like(acc_r)
    acc_p[...] = jnp.zeros_like(acc_p)
    cnt[...] = jnp.zeros_like(cnt)

  deg = d0[...][:, :1] + d1[...][:, :1]
  rdeg = 1.0 / jnp.maximum(deg, 1.0)
  h2 = jnp.maximum((s0[...] + s1[...]) * rdeg + v2[...], 0.0)
  hp = h2 * pcol[...]

  ids = i * RB + lax.broadcasted_iota(jnp.int32, (1, RB), 1)
  onehot_r = (root_ref[...] == ids).astype(jnp.float32)
  acc_r[...] += jnp.dot(onehot_r, h2, preferred_element_type=jnp.float32)

  gids = lax.broadcasted_iota(jnp.int32, (G, 1), 0)
  onehot_b = (batch_ref[...].reshape(1, RB) == gids).astype(jnp.float32)
  acc_p[...] += jnp.dot(onehot_b, hp, preferred_element_type=jnp.float32)
  cnt[...] += jnp.sum(onehot_b, axis=1, keepdims=True)

  @pl.when(i == pl.num_programs(0) - 1)
  def _():
    pooled = acc_p[...] / jnp.maximum(cnt[...], 1.0)
    out_ref[...] = (jnp.dot(acc_r[...], wla[...], preferred_element_type=jnp.float32)
                    + jnp.dot(pooled, wlb[...], preferred_element_type=jnp.float32)
                    + blin_ref[...])


def _row_spec(i):
  return (i, 0)


def _fixed_spec(i):
  return (0, 0)


_tc1 = pl.pallas_call(
    _tc1_body,
    grid=(NPAD // RB,),
    in_specs=[
        pl.BlockSpec((RB, D), _row_spec),
        pl.BlockSpec((D, D), _fixed_spec),
        pl.BlockSpec((D, D), _fixed_spec),
        pl.BlockSpec((1, D), _fixed_spec),
    ],
    out_specs=[pl.BlockSpec((RB, D), _row_spec), pl.BlockSpec((RB, D), _row_spec)],
    out_shape=[jax.ShapeDtypeStruct((NPAD, D), jnp.float32),
               jax.ShapeDtypeStruct((NPAD, D), jnp.float32)],
)

_tc2 = pl.pallas_call(
    _tc2_body,
    grid=(NPAD // RB,),
    in_specs=[
        pl.BlockSpec((RB, D), _row_spec),
        pl.BlockSpec((RB, D), _row_spec),
        pl.BlockSpec((RB, D), _row_spec),
        pl.BlockSpec((RB, D), _row_spec),
        pl.BlockSpec((RB, D), _row_spec),
        pl.BlockSpec((RB, 1), _row_spec),
        pl.BlockSpec((D, D), _fixed_spec),
        pl.BlockSpec((D, D), _fixed_spec),
        pl.BlockSpec((1, D), _fixed_spec),
    ],
    out_specs=[pl.BlockSpec((RB, D), _row_spec), pl.BlockSpec((RB, D), _row_spec)],
    out_shape=[jax.ShapeDtypeStruct((NPAD, D), jnp.float32),
               jax.ShapeDtypeStruct((NPAD, D), jnp.float32)],
)

_tc3 = pl.pallas_call(
    _tc3_body,
    grid=(NPAD // RB,),
    in_specs=[
        pl.BlockSpec((RB, D), _row_spec),
        pl.BlockSpec((RB, D), _row_spec),
        pl.BlockSpec((RB, D), _row_spec),
        pl.BlockSpec((RB, D), _row_spec),
        pl.BlockSpec((RB, D), _row_spec),
        pl.BlockSpec((RB, 1), _row_spec),
        pl.BlockSpec((1, 1, RB), lambda i: (i, 0, 0)),
        pl.BlockSpec((G, 1), _fixed_spec),
        pl.BlockSpec((D, D), _fixed_spec),
        pl.BlockSpec((D, D), _fixed_spec),
        pl.BlockSpec((1, D), _fixed_spec),
    ],
    out_specs=pl.BlockSpec((G, D), _fixed_spec),
    out_shape=jax.ShapeDtypeStruct((G, D), jnp.float32),
    scratch_shapes=[
        pltpu.VMEM((G, D), jnp.float32),
        pltpu.VMEM((G, D), jnp.float32),
        pltpu.VMEM((G, 1), jnp.float32),
    ],
)


def kernel(x, adj_t, root_ptr, p, batch, group_ptr,
           Wl1, bl1, Wr1, Wl2, bl2, Wr2, Wlin, blin):
  del group_ptr
  i32 = jnp.int32
  src = adj_t[0].astype(i32)
  dst = adj_t[1].astype(i32)
  npad = NPAD - N
  epad = EPAD - E

  x_pad = jnp.pad(x, ((0, npad), (0, 0)))
  p_col = jnp.pad(p, (0, npad)).reshape(NPAD, 1)
  batch3d = jnp.pad(batch.astype(i32), (0, npad), constant_values=G).reshape(
      NPAD // RB, 1, RB)
  root_col = root_ptr.astype(i32).reshape(G, 1)

  pad_idx = jnp.full((epad,), N, i32)
  src2d = jnp.concatenate([src, pad_idx]).reshape(EPAD // BATCH, BATCH)
  dst2d = jnp.concatenate([dst, pad_idx]).reshape(EPAD // BATCH, BATCH)

  u1, v1 = _tc1(x_pad, Wl1.T, Wr1.T, bl1.reshape(1, D))
  s1, dg = _edge_deg(u1, src2d, dst2d)
  u2, v2 = _tc2(s1[0], s1[1], dg[0], dg[1], v1, p_col,
                Wl2.T, Wr2.T, bl2.reshape(1, D))
  (s2,) = _edge(u2, src2d, dst2d)
  out = _tc3(s2[0], s2[1], dg[0], dg[1], v2, p_col, batch3d, root_col,
             Wlin[:, :D].T, Wlin[:, D:].T, blin.reshape(1, D))
  return out


# speedup vs baseline: 3.0464x; 1.1734x over previous
import functools

import jax
import jax.numpy as jnp
from jax import lax
from jax.experimental import pallas as pl
from jax.experimental.pallas import tpu as pltpu
from jax.experimental.pallas import tpu_sc as plsc

N = 10000
E = 320000
D = 128
G = 256

NC = 2
NS = 16
LANES = 16
NW = NC * NS

NPAD = 10240
ROWS_PER_TILE = NPAD // NS
EPAD = 327680
EW = EPAD // NW
BATCH = 128
SUPER = 8
NSUPER = EW // (SUPER * BATCH)

RB = 512
DW = 16


def _zero_vmem(ref, rows, cols):
  z = jnp.zeros((LANES,), jnp.float32)
  def body(r, carry):
    for c in range(cols // LANES):
      ref[r, pl.ds(c * LANES, LANES)] = z
    return carry
  lax.fori_loop(0, rows, body, 0)


def _fill_ones_vmem(ref, rows, cols):
  o = jnp.ones((LANES,), jnp.float32)
  def body(r, carry):
    for c in range(cols // LANES):
      ref[r, pl.ds(c * LANES, LANES)] = o
    return carry
  lax.fori_loop(0, rows, body, 0)


def _make_edge_kernel(with_deg):
  mesh = plsc.VectorSubcoreMesh(core_axis_name="c", subcore_axis_name="s")
  out_type = [jax.ShapeDtypeStruct((NC, NPAD, D), jnp.float32)]
  if with_deg:
    out_type.append(jax.ShapeDtypeStruct((NC, NPAD, D), jnp.float32))
  scratch = [
      pltpu.VMEM((SUPER, BATCH), jnp.int32),
      pltpu.VMEM((SUPER, BATCH), jnp.int32),
      pltpu.VMEM((BATCH, D), jnp.float32),
      pltpu.VMEM((BATCH, D), jnp.float32),
      pltpu.VMEM_SHARED((NPAD, D), jnp.float32),
      pltpu.SemaphoreType.DMA,
      pltpu.SemaphoreType.DMA,
      pltpu.SemaphoreType.DMA,
  ]

  def body(table, src2d, dst2d, agg_out, *rest):
    if with_deg:
      deg_out = rest[0]
      rest = rest[1:]
    (sslab, dslab, rows0, rows1, agg_sh, sg0, sg1, ss) = rest
    rows = (rows0, rows1)
    sg = (sg0, sg1)
    cid = lax.axis_index("c")
    sid = lax.axis_index("s")
    wid = sid * NC + cid
    row0 = wid * (EW // BATCH)

    def zero_table():
      for k in range(ROWS_PER_TILE // BATCH):
        pltpu.sync_copy(rows0, agg_sh.at[pl.ds(sid * ROWS_PER_TILE + k * BATCH, BATCH)])

    def dump_table(out):
      for k in range(ROWS_PER_TILE // BATCH):
        r = sid * ROWS_PER_TILE + k * BATCH
        pltpu.sync_copy(agg_sh.at[pl.ds(r, BATCH)], out.at[cid, pl.ds(r, BATCH)])

    if with_deg:
      _zero_vmem(rows0, BATCH, D)
      zero_table()
      _fill_ones_vmem(rows0, BATCH, D)
      plsc.subcore_barrier()

      def deg_block(b, carry):
        pltpu.sync_copy(dst2d.at[pl.ds(row0 + b * SUPER, SUPER)], dslab)
        descs = [pltpu.async_copy(rows0, agg_sh.at[dslab.at[j]], ss, add=True)
                 for j in range(SUPER)]
        for de in descs:
          de.wait()
        return carry

      lax.fori_loop(0, NSUPER, deg_block, 0)
      plsc.subcore_barrier()
      dump_table(deg_out)

    _zero_vmem(rows0, BATCH, D)
    zero_table()
    plsc.subcore_barrier()

    def block(b, carry):
      r = row0 + b * SUPER
      pltpu.sync_copy(src2d.at[pl.ds(r, SUPER)], sslab)
      pltpu.sync_copy(dst2d.at[pl.ds(r, SUPER)], dslab)
      g = {0: pltpu.async_copy(table.at[sslab.at[0]], rows[0], sg[0])}
      for j in range(SUPER - 1):
        g[j].wait()
        g[j + 1] = pltpu.async_copy(table.at[sslab.at[j + 1]],
                                    rows[(j + 1) % 2], sg[(j + 1) % 2])
        pltpu.sync_copy(rows[j % 2], agg_sh.at[dslab.at[j]], add=True)
      g[SUPER - 1].wait()
      pltpu.sync_copy(rows[(SUPER - 1) % 2], agg_sh.at[dslab.at[SUPER - 1]], add=True)
      return carry

    lax.fori_loop(0, NSUPER, block, 0)
    plsc.subcore_barrier()
    dump_table(agg_out)

  return pl.kernel(body, out_type=out_type, mesh=mesh, scratch_types=scratch)


_edge_deg = _make_edge_kernel(True)
_edge = _make_edge_kernel(False)


def _tc1_body(x_ref, wl_ref, wr_ref, bl_ref, u_ref, v_ref):
  xb = x_ref[...]
  u_ref[...] = jnp.dot(xb, wl_ref[...], preferred_element_type=jnp.float32)
  v_ref[...] = jnp.dot(xb, wr_ref[...], preferred_element_type=jnp.float32) + bl_ref[...]


def _tc2_body(s0, s1, d0, d1, v1, pcol, wl, wr, bl, u2, v2):
  deg = d0[...][:, :1] + d1[...][:, :1]
  rdeg = 1.0 / jnp.maximum(deg, 1.0)
  h1 = jnp.maximum((s0[...] + s1[...]) * rdeg + v1[...], 0.0)
  xs = h1 * pcol[...]
  u2[...] = jnp.dot(xs, wl[...], preferred_element_type=jnp.float32)
  v2[...] = jnp.dot(xs, wr[...], preferred_element_type=jnp.float32) + bl[...]


def _tc3_body(s0, s1, d0, d1, v2, pcol, batch_ref, root_ref, wla, wlb, blin_ref,
              out_ref, acc_r, acc_p, cnt):
  i = pl.program_id(0)

  @pl.when(i == 0)
  def _():
    acc_r[...] = jnp.zeros_like(acc_r)
    acc_p[...] = jnp.zeros_like(acc_p)
    cnt[...] = jnp.zeros_like(cnt)

  deg = d0[...][:, :1] + d1[...][:, :1]
  rdeg = 1.0 / jnp.maximum(deg, 1.0)
  h2 = jnp.maximum((s0[...] + s1[...]) * rdeg + v2[...], 0.0)
  hp = h2 * pcol[...]

  ids = i * RB + lax.broadcasted_iota(jnp.int32, (1, RB), 1)
  onehot_r = (root_ref[...] == ids).astype(jnp.float32)
  acc_r[...] += jnp.dot(onehot_r, h2, preferred_element_type=jnp.float32)

  gids = lax.broadcasted_iota(jnp.int32, (G, 1), 0)
  onehot_b = (batch_ref[...].reshape(1, RB) == gids).astype(jnp.float32)
  acc_p[...] += jnp.dot(onehot_b, hp, preferred_element_type=jnp.float32)
  cnt[...] += jnp.sum(onehot_b, axis=1, keepdims=True)

  @pl.when(i == pl.num_programs(0) - 1)
  def _():
    pooled = acc_p[...] / jnp.maximum(cnt[...], 1.0)
    out_ref[...] = (jnp.dot(acc_r[...], wla[...], preferred_element_type=jnp.float32)
                    + jnp.dot(pooled, wlb[...], preferred_element_type=jnp.float32)
                    + blin_ref[...])


def _row_spec(i):
  return (i, 0)


def _fixed_spec(i):
  return (0, 0)


_tc1 = pl.pallas_call(
    _tc1_body,
    grid=(NPAD // RB,),
    in_specs=[
        pl.BlockSpec((RB, D), _row_spec),
        pl.BlockSpec((D, D), _fixed_spec),
        pl.BlockSpec((D, D), _fixed_spec),
        pl.BlockSpec((1, D), _fixed_spec),
    ],
    out_specs=[pl.BlockSpec((RB, D), _row_spec), pl.BlockSpec((RB, D), _row_spec)],
    out_shape=[jax.ShapeDtypeStruct((NPAD, D), jnp.float32),
               jax.ShapeDtypeStruct((NPAD, D), jnp.float32)],
)

_tc2 = pl.pallas_call(
    _tc2_body,
    grid=(NPAD // RB,),
    in_specs=[
        pl.BlockSpec((RB, D), _row_spec),
        pl.BlockSpec((RB, D), _row_spec),
        pl.BlockSpec((RB, D), _row_spec),
        pl.BlockSpec((RB, D), _row_spec),
        pl.BlockSpec((RB, D), _row_spec),
        pl.BlockSpec((RB, 1), _row_spec),
        pl.BlockSpec((D, D), _fixed_spec),
        pl.BlockSpec((D, D), _fixed_spec),
        pl.BlockSpec((1, D), _fixed_spec),
    ],
    out_specs=[pl.BlockSpec((RB, D), _row_spec), pl.BlockSpec((RB, D), _row_spec)],
    out_shape=[jax.ShapeDtypeStruct((NPAD, D), jnp.float32),
               jax.ShapeDtypeStruct((NPAD, D), jnp.float32)],
)

_tc3 = pl.pallas_call(
    _tc3_body,
    grid=(NPAD // RB,),
    in_specs=[
        pl.BlockSpec((RB, D), _row_spec),
        pl.BlockSpec((RB, D), _row_spec),
        pl.BlockSpec((RB, D), _row_spec),
        pl.BlockSpec((RB, D), _row_spec),
        pl.BlockSpec((RB, D), _row_spec),
        pl.BlockSpec((RB, 1), _row_spec),
        pl.BlockSpec((1, 1, RB), lambda i: (i, 0, 0)),
        pl.BlockSpec((G, 1), _fixed_spec),
        pl.BlockSpec((D, D), _fixed_spec),
        pl.BlockSpec((D, D), _fixed_spec),
        pl.BlockSpec((1, D), _fixed_spec),
    ],
    out_specs=pl.BlockSpec((G, D), _fixed_spec),
    out_shape=jax.ShapeDtypeStruct((G, D), jnp.float32),
    scratch_shapes=[
        pltpu.VMEM((G, D), jnp.float32),
        pltpu.VMEM((G, D), jnp.float32),
        pltpu.VMEM((G, 1), jnp.float32),
    ],
)


def kernel(x, adj_t, root_ptr, p, batch, group_ptr,
           Wl1, bl1, Wr1, Wl2, bl2, Wr2, Wlin, blin):
  del group_ptr
  i32 = jnp.int32
  src = adj_t[0].astype(i32)
  dst = adj_t[1].astype(i32)
  npad = NPAD - N
  epad = EPAD - E

  x_pad = jnp.pad(x, ((0, npad), (0, 0)))
  p_col = jnp.pad(p, (0, npad)).reshape(NPAD, 1)
  batch3d = jnp.pad(batch.astype(i32), (0, npad), constant_values=G).reshape(
      NPAD // RB, 1, RB)
  root_col = root_ptr.astype(i32).reshape(G, 1)

  pad_idx = jnp.full((epad,), N, i32)
  src2d = jnp.concatenate([src, pad_idx]).reshape(EPAD // BATCH, BATCH)
  dst2d = jnp.concatenate([dst, pad_idx]).reshape(EPAD // BATCH, BATCH)

  u1, v1 = _tc1(x_pad, Wl1.T, Wr1.T, bl1.reshape(1, D))
  s1, dg = _edge_deg(u1, src2d, dst2d)
  u2, v2 = _tc2(s1[0], s1[1], dg[0], dg[1], v1, p_col,
                Wl2.T, Wr2.T, bl2.reshape(1, D))
  (s2,) = _edge(u2, src2d, dst2d)
  out = _tc3(s2[0], s2[1], dg[0], dg[1], v2, p_col, batch3d, root_col,
             Wlin[:, :D].T, Wlin[:, D:].T, blin.reshape(1, D))
  return out
